# baseline (speedup 1.0000x reference)
import functools

import jax
import jax.numpy as jnp
from jax import lax
from jax.experimental import pallas as pl
from jax.experimental.pallas import tpu as pltpu
from jax.experimental.pallas import tpu_sc as plsc

_NC = 2
_NS = 16
_NW = _NC * _NS
_C = 128

_GATHER_DNUMS = lax.GatherDimensionNumbers(
    offset_dims=(), collapsed_slice_dims=(0,), start_index_map=(0,))


def _lane_broadcast(v16, k):
    idx = jnp.full((16, 1), k, jnp.int32)
    return lax.gather(v16, idx, _GATHER_DNUMS, (1,),
                      mode=lax.GatherScatterMode.PROMISE_IN_BOUNDS)


def _mm_body(x_ref, w_ref, o_ref):
    o_ref[...] = jnp.dot(x_ref[...], w_ref[...],
                         preferred_element_type=jnp.float32)


def _matmul(x, w):
    return pl.pallas_call(
        _mm_body,
        out_shape=jax.ShapeDtypeStruct((x.shape[0], w.shape[1]), jnp.float32),
    )(x, w)


def _ixz_terms(out, eps):
    half = out.shape[1] // 2
    mean = out[:, :half]
    v = out[:, half:]
    softplus = jnp.maximum(v, 0.0) + jnp.log(1.0 + jnp.exp(-jnp.abs(v)))
    std = softplus + 1e-10
    z = mean + std * eps
    t = 0.5 * (z * z - eps * eps) - jnp.log(std)
    return jnp.sum(t) / out.shape[0]


def _epi1_body(p_ref, b1_ref, w2_ref, eps_ref, x1_ref, h2_ref, ixz_ref):
    n = x1_ref.shape[0]
    x1 = p_ref[0, :n] + p_ref[1, :n] + b1_ref[...][None, :]
    x1_ref[...] = x1
    h2_ref[...] = jnp.dot(x1, w2_ref[...], preferred_element_type=jnp.float32)
    ixz_ref[...] = jnp.full((1, 1), _ixz_terms(x1, eps_ref[...]), jnp.float32)


def _epi2_body(p_ref, b2_ref, eps_ref, x2_ref, ixz_ref):
    n = x2_ref.shape[0]
    x2 = p_ref[0, :n] + p_ref[1, :n] + b2_ref[...][None, :]
    x2_ref[...] = x2
    ixz_ref[...] = jnp.full((1, 1), _ixz_terms(x2, eps_ref[...]), jnp.float32)


def _epilogue1(part, b1, w2, eps, n):
    return pl.pallas_call(
        _epi1_body,
        out_shape=(
            jax.ShapeDtypeStruct((n, part.shape[2]), jnp.float32),
            jax.ShapeDtypeStruct((n, w2.shape[1]), jnp.float32),
            jax.ShapeDtypeStruct((1, 1), jnp.float32),
        ),
    )(part, b1, w2, eps)


def _epilogue2(part, b2, eps, n):
    return pl.pallas_call(
        _epi2_body,
        out_shape=(
            jax.ShapeDtypeStruct((n, part.shape[2]), jnp.float32),
            jax.ShapeDtypeStruct((1, 1), jnp.float32),
        ),
    )(part, b2, eps)


@functools.lru_cache(maxsize=None)
def _make_scatter(n, d, ch, n_h):
    rows_pt = n // _NS
    mesh = plsc.VectorSubcoreMesh(core_axis_name="c", subcore_axis_name="s")

    @functools.partial(
        pl.kernel,
        out_type=jax.ShapeDtypeStruct((_NC, n, d), jnp.float32),
        mesh=mesh,
        compiler_params=pltpu.CompilerParams(needs_layout_passes=False,
                                             use_tc_tiling_on_sc=False),
        scratch_types=[
            pltpu.VMEM((ch, _C), jnp.int32),
            pltpu.VMEM((ch, _C), jnp.int32),
            pltpu.VMEM((ch * _C,), jnp.float32),
            pltpu.VMEM((_C, d), jnp.float32),
            pltpu.VMEM((_C, d), jnp.float32),
            pltpu.VMEM_SHARED((n, d), jnp.float32),
            pltpu.VMEM_SHARED((n_h, d), jnp.float32),
            pltpu.SemaphoreType.DMA,
            pltpu.SemaphoreType.DMA,
        ],
    )
    def scatter_kernel(h_hbm, src_hbm, dst_hbm, w_flat_hbm, z_hbm, out_hbm,
                       src_v, dst_v, w_v, rows0_v, rows1_v, acc_sh, h_sh,
                       sem0, sem1):
        c = lax.axis_index("c")
        s = lax.axis_index("s")
        wid = s * _NC + c
        pltpu.sync_copy(z_hbm, acc_sh.at[pl.ds(s * rows_pt, rows_pt)])
        h_pt = (n_h // _NS) & ~7

        @pl.when(s < _NS - 1)
        def _stage_h():
            pltpu.sync_copy(h_hbm.at[pl.ds(s * h_pt, h_pt)],
                            h_sh.at[pl.ds(s * h_pt, h_pt)])

        @pl.when(s == _NS - 1)
        def _stage_h_last():
            rest = n_h - (_NS - 1) * h_pt
            pltpu.sync_copy(h_hbm.at[pl.ds((_NS - 1) * h_pt, rest)],
                            h_sh.at[pl.ds((_NS - 1) * h_pt, rest)])

        pltpu.sync_copy(src_hbm.at[pl.ds(wid * ch, ch)], src_v)
        pltpu.sync_copy(dst_hbm.at[pl.ds(wid * ch, ch)], dst_v)
        pltpu.sync_copy(w_flat_hbm.at[pl.ds(wid * ch * _C, ch * _C)], w_v)
        plsc.subcore_barrier()

        def scale(buf, j):
            def _blk(b, carry2):
                i0 = b * 16
                w16 = w_v[pl.ds(j * _C + i0, 16)]
                for k in range(16):
                    wspl = _lane_broadcast(w16, k)
                    for g in range(d // 16):
                        buf[i0 + k, pl.ds(g * 16, 16)] = (
                            buf[i0 + k, pl.ds(g * 16, 16)] * wspl)
                return carry2

            lax.fori_loop(0, _C // 16, _blk, 0, unroll=False)

        def step(j, buf, sem):
            pltpu.make_async_copy(h_sh.at[src_v.at[j]], buf, sem).wait()
            scale(buf, j)
            pltpu.sync_copy(buf, acc_sh.at[dst_v.at[j]], add=True)

            @pl.when(j + 2 < ch)
            def _prefetch():
                pltpu.async_copy(h_sh.at[src_v.at[j + 2]], buf, sem)

        pltpu.async_copy(h_sh.at[src_v.at[0]], rows0_v, sem0)
        pltpu.async_copy(h_sh.at[src_v.at[1]], rows1_v, sem1)

        def pair_body(t, carry):
            step(2 * t, rows0_v, sem0)
            step(2 * t + 1, rows1_v, sem1)
            return carry

        lax.fori_loop(0, ch // 2, pair_body, 0, unroll=False)
        plsc.subcore_barrier()
        pltpu.sync_copy(acc_sh.at[pl.ds(s * rows_pt, rows_pt)],
                        out_hbm.at[c, pl.ds(s * rows_pt, rows_pt)])

    return scatter_kernel


def _scatter(h, src2d, dst2d, w_flat, n, d, ch):
    zeros = jnp.zeros((n // _NS, d), jnp.float32)
    return _make_scatter(n, d, ch, h.shape[0])(h, src2d, dst2d, w_flat, zeros)


_N_PAD = 10240


def kernel(x, edge_index, edge_attr, W1, b1, W2, b2):
    n = x.shape[0]
    n_pad = max(_N_PAD, -(-n // (8 * _NS)) * 8 * _NS)
    e = edge_attr.shape[0]
    per = _NW * _C
    ch = -(-(-(-e // per)) // 8) * 8
    e_pad = ch * per
    src = jnp.pad(edge_index[0], (0, e_pad - e)).reshape(_NW * ch, _C)
    dst = jnp.pad(edge_index[1], (0, e_pad - e)).reshape(_NW * ch, _C)
    w = jnp.pad(edge_attr, (0, e_pad - e))

    latent = W1.shape[1]
    classes = W2.shape[1]
    eps1 = jax.random.normal(jax.random.key(1), (n, (latent + 1) // 2),
                             dtype=jnp.float32)
    eps2 = jax.random.normal(jax.random.key(2), (n, (classes + 1) // 2),
                             dtype=jnp.float32)

    h1 = _matmul(x, W1)
    part1 = _scatter(h1, src, dst, w, n_pad, latent, ch)
    x1, h2, ixz1 = _epilogue1(part1, b1, W2, eps1, n)
    part2 = _scatter(h2, src, dst, w, n_pad, classes, ch)
    x2, ixz2 = _epilogue2(part2, b2, eps2, n)

    zero = jnp.zeros([], jnp.float32)
    return (x2, x1, ixz1.reshape(()), ixz2.reshape(()), zero, zero)


# speedup vs baseline: 1.1729x; 1.1729x over previous
import functools

import jax
import jax.numpy as jnp
from jax import lax
from jax.experimental import pallas as pl
from jax.experimental.pallas import tpu as pltpu
from jax.experimental.pallas import tpu_sc as plsc

_NC = 2
_NS = 16
_NW = _NC * _NS
_C = 128

_GATHER_DNUMS = lax.GatherDimensionNumbers(
    offset_dims=(), collapsed_slice_dims=(0,), start_index_map=(0,))


def _lane_broadcast(v16, k):
    idx = jnp.full((16, 1), k, jnp.int32)
    return lax.gather(v16, idx, _GATHER_DNUMS, (1,),
                      mode=lax.GatherScatterMode.PROMISE_IN_BOUNDS)


def _mm_body(x_ref, w_ref, o_ref):
    o_ref[...] = jnp.dot(x_ref[...], w_ref[...],
                         preferred_element_type=jnp.float32)


def _matmul(x, w):
    return pl.pallas_call(
        _mm_body,
        out_shape=jax.ShapeDtypeStruct((x.shape[0], w.shape[1]), jnp.float32),
    )(x, w)


def _ixz_terms(out, eps):
    half = out.shape[1] // 2
    mean = out[:, :half]
    v = out[:, half:]
    softplus = jnp.maximum(v, 0.0) + jnp.log(1.0 + jnp.exp(-jnp.abs(v)))
    std = softplus + 1e-10
    z = mean + std * eps
    t = 0.5 * (z * z - eps * eps) - jnp.log(std)
    return jnp.sum(t) / out.shape[0]


def _epi1_body(p_ref, b1_ref, w2_ref, eps_ref, x1_ref, h2_ref, ixz_ref):
    n = x1_ref.shape[0]
    x1 = p_ref[0, :n] + p_ref[1, :n] + b1_ref[...][None, :]
    x1_ref[...] = x1
    h2_ref[...] = jnp.dot(x1, w2_ref[...], preferred_element_type=jnp.float32)
    ixz_ref[...] = jnp.full((1, 1), _ixz_terms(x1, eps_ref[...]), jnp.float32)


def _epi2_body(p_ref, b2_ref, eps_ref, x2_ref, ixz_ref):
    n = x2_ref.shape[0]
    x2 = p_ref[0, :n] + p_ref[1, :n] + b2_ref[...][None, :]
    x2_ref[...] = x2
    ixz_ref[...] = jnp.full((1, 1), _ixz_terms(x2, eps_ref[...]), jnp.float32)


def _epilogue1(part, b1, w2, eps, n):
    return pl.pallas_call(
        _epi1_body,
        out_shape=(
            jax.ShapeDtypeStruct((n, part.shape[2]), jnp.float32),
            jax.ShapeDtypeStruct((n, w2.shape[1]), jnp.float32),
            jax.ShapeDtypeStruct((1, 1), jnp.float32),
        ),
    )(part, b1, w2, eps)


def _epilogue2(part, b2, eps, n):
    return pl.pallas_call(
        _epi2_body,
        out_shape=(
            jax.ShapeDtypeStruct((n, part.shape[2]), jnp.float32),
            jax.ShapeDtypeStruct((1, 1), jnp.float32),
        ),
    )(part, b2, eps)


@functools.lru_cache(maxsize=None)
def _make_scatter(n, d, ch, n_h):
    rows_pt = n // _NS
    mesh = plsc.VectorSubcoreMesh(core_axis_name="c", subcore_axis_name="s")

    @functools.partial(
        pl.kernel,
        out_type=jax.ShapeDtypeStruct((_NC, n, d), jnp.float32),
        mesh=mesh,
        compiler_params=pltpu.CompilerParams(needs_layout_passes=False,
                                             use_tc_tiling_on_sc=False),
        scratch_types=[
            pltpu.VMEM((ch, _C), jnp.int32),
            pltpu.VMEM((ch, _C), jnp.int32),
            pltpu.VMEM((ch * _C,), jnp.float32),
            pltpu.VMEM((_C, d), jnp.float32),
            pltpu.VMEM((_C, d), jnp.float32),
            pltpu.VMEM_SHARED((n, d), jnp.float32),
            pltpu.VMEM_SHARED((n_h, d), jnp.float32),
            pltpu.SemaphoreType.DMA,
            pltpu.SemaphoreType.DMA,
        ],
    )
    def scatter_kernel(h_hbm, src_hbm, dst_hbm, w_flat_hbm, z_hbm, out_hbm,
                       src_v, dst_v, w_v, rows0_v, rows1_v, acc_sh, h_sh,
                       sem0, sem1):
        c = lax.axis_index("c")
        s = lax.axis_index("s")
        wid = s * _NC + c
        pltpu.sync_copy(z_hbm, acc_sh.at[pl.ds(s * rows_pt, rows_pt)])
        h_pt = (n_h // _NS) & ~7

        @pl.when(s < _NS - 1)
        def _stage_h():
            pltpu.sync_copy(h_hbm.at[pl.ds(s * h_pt, h_pt)],
                            h_sh.at[pl.ds(s * h_pt, h_pt)])

        @pl.when(s == _NS - 1)
        def _stage_h_last():
            rest = n_h - (_NS - 1) * h_pt
            pltpu.sync_copy(h_hbm.at[pl.ds((_NS - 1) * h_pt, rest)],
                            h_sh.at[pl.ds((_NS - 1) * h_pt, rest)])

        pltpu.sync_copy(src_hbm.at[pl.ds(wid * ch, ch)], src_v)
        pltpu.sync_copy(dst_hbm.at[pl.ds(wid * ch, ch)], dst_v)
        pltpu.sync_copy(w_flat_hbm.at[pl.ds(wid * ch * _C, ch * _C)], w_v)
        plsc.subcore_barrier()

        def scale(buf, j):
            def _rows(i, carry2):
                wspl = plsc.load_gather(
                    w_v, [jnp.full((16,), j * _C + i, jnp.int32)])
                for g in range(d // 16):
                    buf[i, pl.ds(g * 16, 16)] = buf[i, pl.ds(g * 16, 16)] * wspl
                return carry2

            lax.fori_loop(0, _C, _rows, 0, unroll=16)

        def step(j, buf, sem):
            pltpu.make_async_copy(h_sh.at[src_v.at[j]], buf, sem).wait()
            scale(buf, j)
            pltpu.sync_copy(buf, acc_sh.at[dst_v.at[j]], add=True)

            @pl.when(j + 2 < ch)
            def _prefetch():
                pltpu.async_copy(h_sh.at[src_v.at[j + 2]], buf, sem)

        pltpu.async_copy(h_sh.at[src_v.at[0]], rows0_v, sem0)
        pltpu.async_copy(h_sh.at[src_v.at[1]], rows1_v, sem1)

        def pair_body(t, carry):
            step(2 * t, rows0_v, sem0)
            step(2 * t + 1, rows1_v, sem1)
            return carry

        lax.fori_loop(0, ch // 2, pair_body, 0, unroll=False)
        plsc.subcore_barrier()
        pltpu.sync_copy(acc_sh.at[pl.ds(s * rows_pt, rows_pt)],
                        out_hbm.at[c, pl.ds(s * rows_pt, rows_pt)])

    return scatter_kernel


def _scatter(h, src2d, dst2d, w_flat, n, d, ch):
    zeros = jnp.zeros((n // _NS, d), jnp.float32)
    return _make_scatter(n, d, ch, h.shape[0])(h, src2d, dst2d, w_flat, zeros)


_N_PAD = 10240


def kernel(x, edge_index, edge_attr, W1, b1, W2, b2):
    n = x.shape[0]
    n_pad = max(_N_PAD, -(-n // (8 * _NS)) * 8 * _NS)
    e = edge_attr.shape[0]
    per = _NW * _C
    ch = -(-(-(-e // per)) // 8) * 8
    e_pad = ch * per
    src = jnp.pad(edge_index[0], (0, e_pad - e)).reshape(_NW * ch, _C)
    dst = jnp.pad(edge_index[1], (0, e_pad - e)).reshape(_NW * ch, _C)
    w = jnp.pad(edge_attr, (0, e_pad - e))

    latent = W1.shape[1]
    classes = W2.shape[1]
    eps1 = jax.random.normal(jax.random.key(1), (n, (latent + 1) // 2),
                             dtype=jnp.float32)
    eps2 = jax.random.normal(jax.random.key(2), (n, (classes + 1) // 2),
                             dtype=jnp.float32)

    h1 = _matmul(x, W1)
    part1 = _scatter(h1, src, dst, w, n_pad, latent, ch)
    x1, h2, ixz1 = _epilogue1(part1, b1, W2, eps1, n)
    part2 = _scatter(h2, src, dst, w, n_pad, classes, ch)
    x2, ixz2 = _epilogue2(part2, b2, eps2, n)

    zero = jnp.zeros([], jnp.float32)
    return (x2, x1, ixz1.reshape(()), ixz2.reshape(()), zero, zero)


# speedup vs baseline: 1.3009x; 1.1092x over previous
import functools

import jax
import jax.numpy as jnp
from jax import lax
from jax.experimental import pallas as pl
from jax.experimental.pallas import tpu as pltpu
from jax.experimental.pallas import tpu_sc as plsc

_NC = 2
_NS = 16
_NW = _NC * _NS
_C = 128

_GATHER_DNUMS = lax.GatherDimensionNumbers(
    offset_dims=(), collapsed_slice_dims=(0,), start_index_map=(0,))


def _lane_broadcast(v16, k):
    idx = jnp.full((16, 1), k, jnp.int32)
    return lax.gather(v16, idx, _GATHER_DNUMS, (1,),
                      mode=lax.GatherScatterMode.PROMISE_IN_BOUNDS)


def _mm_body(x_ref, w_ref, o_ref):
    o_ref[...] = jnp.dot(x_ref[...], w_ref[...],
                         preferred_element_type=jnp.float32)


def _matmul(x, w):
    return pl.pallas_call(
        _mm_body,
        out_shape=jax.ShapeDtypeStruct((x.shape[0], w.shape[1]), jnp.float32),
    )(x, w)


def _ixz_terms(out, eps):
    half = out.shape[1] // 2
    mean = out[:, :half]
    v = out[:, half:]
    softplus = jnp.maximum(v, 0.0) + jnp.log(1.0 + jnp.exp(-jnp.abs(v)))
    std = softplus + 1e-10
    z = mean + std * eps
    t = 0.5 * (z * z - eps * eps) - jnp.log(std)
    return jnp.sum(t) / out.shape[0]


def _epi1_body(p_ref, b1_ref, w2_ref, eps_ref, x1_ref, h2_ref, ixz_ref):
    n = x1_ref.shape[0]
    x1 = p_ref[0, :n] + p_ref[1, :n] + b1_ref[...][None, :]
    x1_ref[...] = x1
    h2_ref[...] = jnp.dot(x1, w2_ref[...], preferred_element_type=jnp.float32)
    ixz_ref[...] = jnp.full((1, 1), _ixz_terms(x1, eps_ref[...]), jnp.float32)


def _epi2_body(p_ref, b2_ref, eps_ref, x2_ref, ixz_ref):
    n = x2_ref.shape[0]
    x2 = p_ref[0, :n] + p_ref[1, :n] + b2_ref[...][None, :]
    x2_ref[...] = x2
    ixz_ref[...] = jnp.full((1, 1), _ixz_terms(x2, eps_ref[...]), jnp.float32)


def _epilogue1(part, b1, w2, eps, n):
    return pl.pallas_call(
        _epi1_body,
        out_shape=(
            jax.ShapeDtypeStruct((n, part.shape[2]), jnp.float32),
            jax.ShapeDtypeStruct((n, w2.shape[1]), jnp.float32),
            jax.ShapeDtypeStruct((1, 1), jnp.float32),
        ),
    )(part, b1, w2, eps)


def _epilogue2(part, b2, eps, n):
    return pl.pallas_call(
        _epi2_body,
        out_shape=(
            jax.ShapeDtypeStruct((n, part.shape[2]), jnp.float32),
            jax.ShapeDtypeStruct((1, 1), jnp.float32),
        ),
    )(part, b2, eps)


@functools.lru_cache(maxsize=None)
def _make_scatter(n, d, ch, n_h):
    rows_pt = n // _NS
    mesh = plsc.VectorSubcoreMesh(core_axis_name="c", subcore_axis_name="s")

    @functools.partial(
        pl.kernel,
        out_type=jax.ShapeDtypeStruct((_NC, n, d), jnp.float32),
        mesh=mesh,
        compiler_params=pltpu.CompilerParams(needs_layout_passes=False,
                                             use_tc_tiling_on_sc=False),
        scratch_types=[
            pltpu.VMEM((ch // 2, _C), jnp.int32),
            pltpu.VMEM((ch // 2, _C), jnp.int32),
            pltpu.VMEM((ch // 2 * _C,), jnp.float32),
            pltpu.VMEM((_C, d), jnp.float32),
            pltpu.VMEM((_C, d), jnp.float32),
            pltpu.VMEM((_C, d), jnp.float32),
            pltpu.VMEM((_C, d), jnp.float32),
            pltpu.VMEM_SHARED((n, d), jnp.float32),
            pltpu.VMEM_SHARED((n_h, d), jnp.float32),
            pltpu.SemaphoreType.DMA,
            pltpu.SemaphoreType.DMA,
            pltpu.SemaphoreType.DMA,
            pltpu.SemaphoreType.DMA,
            pltpu.SemaphoreType.DMA,
            pltpu.SemaphoreType.DMA,
            pltpu.SemaphoreType.DMA,
            pltpu.SemaphoreType.DMA,
        ],
    )
    def scatter_kernel(h_hbm, src_hbm, dst_hbm, w_flat_hbm, z_hbm, out_hbm,
                       src_v, dst_v, w_v, rows0_v, rows1_v, rows2_v, rows3_v,
                       acc_sh, h_sh,
                       gsem0, gsem1, gsem2, gsem3,
                       ssem0, ssem1, ssem2, ssem3):
        c = lax.axis_index("c")
        s = lax.axis_index("s")
        wid = s * _NC + c
        pltpu.sync_copy(z_hbm, acc_sh.at[pl.ds(s * rows_pt, rows_pt)])
        h_pt = (n_h // _NS) & ~7

        @pl.when(s < _NS - 1)
        def _stage_h():
            pltpu.sync_copy(h_hbm.at[pl.ds(s * h_pt, h_pt)],
                            h_sh.at[pl.ds(s * h_pt, h_pt)])

        @pl.when(s == _NS - 1)
        def _stage_h_last():
            rest = n_h - (_NS - 1) * h_pt
            pltpu.sync_copy(h_hbm.at[pl.ds((_NS - 1) * h_pt, rest)],
                            h_sh.at[pl.ds((_NS - 1) * h_pt, rest)])

        plsc.subcore_barrier()
        ch2 = ch // 2

        def scale(buf, j):
            def _rows(i, carry2):
                wspl = plsc.load_gather(
                    w_v, [jnp.full((16,), j * _C + i, jnp.int32)])
                for g in range(d // 16):
                    buf[i, pl.ds(g * 16, 16)] = buf[i, pl.ds(g * 16, 16)] * wspl
                return carry2

            lax.fori_loop(0, _C, _rows, 0, unroll=16)

        bufs = (rows0_v, rows1_v, rows2_v, rows3_v)
        gsems = (gsem0, gsem1, gsem2, gsem3)
        ssems = (ssem0, ssem1, ssem2, ssem3)

        def step(j, b, first):
            pltpu.make_async_copy(h_sh.at[src_v.at[j]], bufs[b], gsems[b]).wait()
            scale(bufs[b], j)
            pltpu.async_copy(bufs[b], acc_sh.at[dst_v.at[j]], ssems[b],
                             add=True)
            b2 = (b + 2) % 4

            @pl.when(j + 2 < ch2)
            def _prefetch():
                def _fire():
                    pltpu.make_async_copy(
                        bufs[b2], acc_sh.at[dst_v.at[j - 2]], ssems[b2]).wait()
                    pltpu.async_copy(h_sh.at[src_v.at[j + 2]], bufs[b2],
                                     gsems[b2])

                if first:
                    pl.when(j >= 2)(_fire)

                    @pl.when(j < 2)
                    def _fire_nowait():
                        pltpu.async_copy(h_sh.at[src_v.at[j + 2]], bufs[b2],
                                         gsems[b2])
                else:
                    _fire()

        def half_pass(half):
            eb = wid * ch + half * ch2
            pltpu.sync_copy(src_hbm.at[pl.ds(eb, ch2)], src_v)
            pltpu.sync_copy(dst_hbm.at[pl.ds(eb, ch2)], dst_v)
            pltpu.sync_copy(w_flat_hbm.at[pl.ds(eb * _C, ch2 * _C)], w_v)
            pltpu.async_copy(h_sh.at[src_v.at[0]], rows0_v, gsem0)
            pltpu.async_copy(h_sh.at[src_v.at[1]], rows1_v, gsem1)

            def quad_body(t, carry):
                step(4 * t, 0, True)
                step(4 * t + 1, 1, True)
                step(4 * t + 2, 2, False)
                step(4 * t + 3, 3, False)
                return carry

            lax.fori_loop(0, ch2 // 4, quad_body, 0, unroll=False)
            for jj in range(ch2 - 4, ch2):
                b = jj % 4
                pltpu.make_async_copy(bufs[b], acc_sh.at[dst_v.at[jj]],
                                      ssems[b]).wait()

        half_pass(0)
        half_pass(1)
        plsc.subcore_barrier()
        pltpu.sync_copy(acc_sh.at[pl.ds(s * rows_pt, rows_pt)],
                        out_hbm.at[c, pl.ds(s * rows_pt, rows_pt)])

    return scatter_kernel


def _scatter(h, src2d, dst2d, w_flat, n, d, ch):
    zeros = jnp.zeros((n // _NS, d), jnp.float32)
    return _make_scatter(n, d, ch, h.shape[0])(h, src2d, dst2d, w_flat, zeros)


_N_PAD = 10240


def kernel(x, edge_index, edge_attr, W1, b1, W2, b2):
    n = x.shape[0]
    n_pad = max(_N_PAD, -(-n // (8 * _NS)) * 8 * _NS)
    e = edge_attr.shape[0]
    per = _NW * _C
    ch = -(-(-(-e // per)) // 8) * 8
    e_pad = ch * per
    src = jnp.pad(edge_index[0], (0, e_pad - e)).reshape(_NW * ch, _C)
    dst = jnp.pad(edge_index[1], (0, e_pad - e)).reshape(_NW * ch, _C)
    w = jnp.pad(edge_attr, (0, e_pad - e))

    latent = W1.shape[1]
    classes = W2.shape[1]
    eps1 = jax.random.normal(jax.random.key(1), (n, (latent + 1) // 2),
                             dtype=jnp.float32)
    eps2 = jax.random.normal(jax.random.key(2), (n, (classes + 1) // 2),
                             dtype=jnp.float32)

    h1 = _matmul(x, W1)
    part1 = _scatter(h1, src, dst, w, n_pad, latent, ch)
    x1, h2, ixz1 = _epilogue1(part1, b1, W2, eps1, n)
    part2 = _scatter(h2, src, dst, w, n_pad, classes, ch)
    x2, ixz2 = _epilogue2(part2, b2, eps2, n)

    zero = jnp.zeros([], jnp.float32)
    return (x2, x1, ixz1.reshape(()), ixz2.reshape(()), zero, zero)
